# Initial kernel scaffold; baseline (speedup 1.0000x reference)
import jax
import jax.numpy as jnp
from jax.experimental import pallas as pl


def kernel(x, edge_index, h, W1, b1, W_ih, W_hh, b_ih, b_hh, Wg, bg, Wa, ba):
    raise NotImplementedError("write your pallas kernel here")


# speedup vs baseline: 5.0031x; 5.0031x over previous
import functools

import jax
import jax.numpy as jnp
from jax import lax
from jax.experimental import pallas as pl
from jax.experimental.pallas import tpu as pltpu
from jax.experimental.pallas import tpu_sc as plsc

_N = 10000
_E = 320000
_H = 128

_NC = 2
_NS = 16
_NW = _NC * _NS
_EPW = _E // _NW
_C = 80
_NCHUNK = _EPW // _C
_NPAD = 10240
_ZCH = _NPAD // _C
_ZPS = _ZCH // _NS

_BLK = 1000


def _mlp_body(x_ref, w_ref, b_ref, y_ref):
    y = jnp.dot(x_ref[...], w_ref[...], preferred_element_type=jnp.float32)
    y_ref[...] = jnp.maximum(y + b_ref[...], 0.0)


def _node_mlp(x, w1t, b1):
    return pl.pallas_call(
        _mlp_body,
        grid=(_N // _BLK,),
        in_specs=[
            pl.BlockSpec((_BLK, _H), lambda i: (i, 0)),
            pl.BlockSpec((_H, _H), lambda i: (0, 0)),
            pl.BlockSpec((1, _H), lambda i: (0, 0)),
        ],
        out_specs=pl.BlockSpec((_BLK, _H), lambda i: (i, 0)),
        out_shape=jax.ShapeDtypeStruct((_N, _H), jnp.float32),
    )(x, w1t, b1)


def _seg_sum_body(y_hbm, row_hbm, col_hbm, out_hbm, ridx, cidx, rows, acc, sem):
    c = lax.axis_index("c")
    s = lax.axis_index("s")
    wid = s * _NC + c

    zero16 = jnp.zeros((16,), jnp.float32)

    def zrows(i, carry):
        rows[i // (_H // 16), pl.ds((i % (_H // 16)) * 16, 16)] = zero16
        return carry

    lax.fori_loop(0, _C * (_H // 16), zrows, 0)

    def zacc(k, carry):
        j = s * _ZPS + k
        pltpu.sync_copy(rows, acc.at[pl.ds(j * _C, _C)])
        return carry

    lax.fori_loop(0, _ZPS, zacc, 0)
    plsc.subcore_barrier()

    base = wid * _EPW

    def ebody(j, carry):
        off = pl.multiple_of(base + j * _C, 8)
        pltpu.sync_copy(row_hbm.at[pl.ds(off, _C)], ridx)
        pltpu.sync_copy(col_hbm.at[pl.ds(off, _C)], cidx)
        pltpu.async_copy(y_hbm.at[ridx], rows, sem).wait()
        pltpu.sync_copy(rows, acc.at[cidx], add=True)
        return carry

    lax.fori_loop(0, _NCHUNK, ebody, 0)
    plsc.subcore_barrier()

    def wout(k, carry):
        j = s * _ZPS + k
        pltpu.sync_copy(acc.at[pl.ds(j * _C, _C)], rows)
        pltpu.sync_copy(rows, out_hbm.at[c, pl.ds(j * _C, _C)])
        return carry

    lax.fori_loop(0, _ZPS, wout, 0)


def _seg_sum_sc(y, row, col):
    mesh = plsc.VectorSubcoreMesh(
        core_axis_name="c", subcore_axis_name="s",
        num_cores=_NC, num_subcores=_NS)
    f = functools.partial(
        pl.kernel,
        mesh=mesh,
        out_type=jax.ShapeDtypeStruct((_NC, _NPAD, _H), jnp.float32),
        scratch_types=[
            pltpu.VMEM((_C,), jnp.int32),
            pltpu.VMEM((_C,), jnp.int32),
            pltpu.VMEM((_C, _H), jnp.float32),
            pltpu.VMEM_SHARED((_NPAD, _H), jnp.float32),
            pltpu.SemaphoreType.DMA,
        ],
    )(_seg_sum_body)
    return f(y, row, col)


def _gru_head_body(p_ref, x_ref, h_ref, wih_ref, whh_ref, bih_ref, bhh_ref,
                   wg_ref, bg_ref, wax_ref, wag_ref, ba_ref, a_ref, hn_ref):
    xt = p_ref[0] + p_ref[1]
    h0 = h_ref[...]
    gi = jnp.dot(xt, wih_ref[...], preferred_element_type=jnp.float32) + bih_ref[...]
    gh = jnp.dot(h0, whh_ref[...], preferred_element_type=jnp.float32) + bhh_ref[...]
    r = jax.nn.sigmoid(gi[:, :_H] + gh[:, :_H])
    z = jax.nn.sigmoid(gi[:, _H:2 * _H] + gh[:, _H:2 * _H])
    n = jnp.tanh(gi[:, 2 * _H:] + r * gh[:, 2 * _H:])
    hn = (1.0 - z) * n + z * h0
    hn_ref[...] = hn
    g = jnp.maximum(
        jnp.dot(hn, wg_ref[...], preferred_element_type=jnp.float32) + bg_ref[...], 0.0)
    sacc = (jnp.dot(x_ref[...], wax_ref[...], preferred_element_type=jnp.float32)
            + jnp.dot(g, wag_ref[...], preferred_element_type=jnp.float32)
            + ba_ref[...])
    a_ref[...] = jax.nn.softplus(sacc)


def _gru_head(p, x, h, wiht, whht, bih, bhh, wgt, bg, waxt, wagt, ba):
    return pl.pallas_call(
        _gru_head_body,
        grid=(_N // _BLK,),
        in_specs=[
            pl.BlockSpec((_NC, _BLK, _H), lambda i: (0, i, 0)),
            pl.BlockSpec((_BLK, _H), lambda i: (i, 0)),
            pl.BlockSpec((_BLK, _H), lambda i: (i, 0)),
            pl.BlockSpec((_H, 3 * _H), lambda i: (0, 0)),
            pl.BlockSpec((_H, 3 * _H), lambda i: (0, 0)),
            pl.BlockSpec((1, 3 * _H), lambda i: (0, 0)),
            pl.BlockSpec((1, 3 * _H), lambda i: (0, 0)),
            pl.BlockSpec((_H, _H), lambda i: (0, 0)),
            pl.BlockSpec((1, _H), lambda i: (0, 0)),
            pl.BlockSpec((_H, 1), lambda i: (0, 0)),
            pl.BlockSpec((_H, 1), lambda i: (0, 0)),
            pl.BlockSpec((1, 1), lambda i: (0, 0)),
        ],
        out_specs=[
            pl.BlockSpec((_BLK, 1), lambda i: (i, 0)),
            pl.BlockSpec((_BLK, _H), lambda i: (i, 0)),
        ],
        out_shape=[
            jax.ShapeDtypeStruct((_N, 1), jnp.float32),
            jax.ShapeDtypeStruct((_N, _H), jnp.float32),
        ],
    )(p, x, h, wiht, whht, bih, bhh, wgt, bg, waxt, wagt, ba)


def kernel(x, edge_index, h, W1, b1, W_ih, W_hh, b_ih, b_hh, Wg, bg, Wa, ba):
    row = edge_index[0]
    col = edge_index[1]
    y = _node_mlp(x, W1.T, b1.reshape(1, _H))
    p = _seg_sum_sc(y, row, col)
    a, h_new = _gru_head(
        p, x, h,
        W_ih.T, W_hh.T, b_ih.reshape(1, 3 * _H), b_hh.reshape(1, 3 * _H),
        Wg.T, bg.reshape(1, _H),
        Wa[:, :_H].T, Wa[:, _H:].T, ba.reshape(1, 1))
    return (a, h_new)
